# baseline (speedup 1.0000x reference)
import functools

import jax
import jax.numpy as jnp
from jax import lax
from jax.experimental import pallas as pl
from jax.experimental.pallas import tpu as pltpu
from jax.experimental.pallas import tpu_sc as plsc

_NC = 2
_NS = 16
_NW = _NC * _NS
_LANES = 16
_EB = 128


def _sc_edge_kernel(n_nodes, n_node_pad, n_pad, n_edges, cb):
    nb = n_edges // _EB
    base_nb = nb // _NW
    extra = nb - base_nb * _NW
    n_chunks = base_nb // cb
    assert base_nb == n_chunks * cb
    chunk = cb * _EB
    n2 = 2 * n_pad

    mesh = plsc.VectorSubcoreMesh(core_axis_name="c", subcore_axis_name="s")

    in_bufs = (
        pltpu.VMEM((chunk,), jnp.int32),
        pltpu.VMEM((chunk,), jnp.int32),
        pltpu.VMEM((cb, 4, _EB), jnp.float32),
        pltpu.SemaphoreType.DMA,
    )
    out_bufs = (
        pltpu.VMEM((2 * chunk,), jnp.float32),
        pltpu.VMEM((2 * chunk,), jnp.float32),
        pltpu.SemaphoreType.DMA,
    )

    @functools.partial(
        pl.kernel,
        mesh=mesh,
        compiler_params=pltpu.CompilerParams(needs_layout_passes=False),
        out_type=(
            jax.ShapeDtypeStruct((2 * n_edges,), jnp.float32),
            jax.ShapeDtypeStruct((2 * n_edges,), jnp.float32),
            jax.ShapeDtypeStruct((_NW * n2,), jnp.float32),
        ),
        name="sc_edges",
        scratch_types=(
            pltpu.VMEM((n_node_pad,), jnp.float32),
            pltpu.VMEM((n_node_pad,), jnp.float32),
            pltpu.VMEM((n2,), jnp.float32),
            pltpu.SemaphoreType.DMA,
        ) + in_bufs + in_bufs + out_bufs,
    )
    def sc_kernel(vre_hbm, vim_hbm, zeros_hbm, ef_hbm, s_hbm, r_hbm,
                  i_hbm, v_hbm, part_hbm,
                  vre_buf, vim_buf, acc_buf, v2_sem, *bufs):
        in_sets = (bufs[0:4], bufs[4:8])
        out_set = bufs[8:11]
        wid = lax.axis_index("s") * _NC + lax.axis_index("c")
        iota = lax.iota(jnp.int32, _LANES)
        start_block = wid * base_nb + jnp.minimum(wid, extra)
        total = n_chunks

        v2_d = [pltpu.async_copy(vre_hbm, vre_buf, v2_sem),
                pltpu.async_copy(vim_hbm, vim_buf, v2_sem),
                pltpu.async_copy(zeros_hbm, acc_buf, v2_sem)]

        def issue_in(c, bset):
            s_buf, r_buf, ef_buf, sem = bset
            bstart = start_block + c * cb
            ebase = bstart * _EB
            return [
                pltpu.async_copy(s_hbm.at[pl.ds(ebase, chunk)], s_buf, sem),
                pltpu.async_copy(r_hbm.at[pl.ds(ebase, chunk)], r_buf, sem),
                pltpu.async_copy(ef_hbm.at[pl.ds(bstart, cb)], ef_buf, sem),
            ]

        def issue_out(c):
            iout_buf, vout_buf, sem = out_set
            bstart = start_block + c * cb
            return [
                pltpu.async_copy(iout_buf,
                                 i_hbm.at[pl.ds(bstart * 2 * _EB, 2 * chunk)],
                                 sem),
                pltpu.async_copy(vout_buf,
                                 v_hbm.at[pl.ds(bstart * 2 * _EB, 2 * chunk)],
                                 sem),
            ]

        def make_step(bset, nsteps):
            s_buf, r_buf, ef_buf = bset[:3]
            iout_buf, vout_buf = out_set[:2]

            @plsc.parallel_loop(0, nsteps * _LANES, step=_LANES, unroll=4)
            def step(off):
                blk = off // _EB
                rem = off - blk * _EB
                s = s_buf[pl.ds(off, _LANES)]
                r = r_buf[pl.ds(off, _LANES)]
                g = ef_buf[blk, 0, pl.ds(rem, _LANES)]
                bb = ef_buf[blk, 1, pl.ds(rem, _LANES)]
                vr_re = plsc.load_gather(vre_buf, [r])
                vr_im = plsc.load_gather(vim_buf, [r])
                vs_re = plsc.load_gather(vre_buf, [s])
                vs_im = plsc.load_gather(vim_buf, [s])
                v_re = vr_re - vs_re
                v_im = vr_im - vs_im
                i_re = g * v_re - bb * v_im
                i_im = g * v_im + bb * v_re
                pos = blk * 2 * _EB + rem
                vout_buf[pl.ds(pos, _LANES)] = v_re
                vout_buf[pl.ds(pos + _EB, _LANES)] = v_im
                iout_buf[pl.ds(pos, _LANES)] = i_re
                iout_buf[pl.ds(pos + _EB, _LANES)] = i_im
                plsc.addupdate_scatter(acc_buf, [r], i_re)
                plsc.addupdate_scatter(acc_buf, [r + n_pad], i_im)
                plsc.addupdate_scatter(acc_buf, [s], -i_re)
                plsc.addupdate_scatter(acc_buf, [s + n_pad], -i_im)

        in_d = {c: issue_in(c, in_sets[c]) for c in range(total)}
        for d in v2_d:
            d.wait()

        out_d = []
        for c in range(total):
            for d in in_d.pop(c):
                d.wait()
            for d in out_d:
                d.wait()
            make_step(in_sets[c], chunk // _LANES)
            out_d = issue_out(c)

        for d in out_d:
            d.wait()

        @pl.when(wid < extra)
        def _():
            s_buf, r_buf, ef_buf = in_sets[0][:3]
            iout_buf, vout_buf = out_set[:2]
            bstart = start_block + base_nb
            ebase = bstart * _EB
            pltpu.sync_copy(s_hbm.at[pl.ds(ebase, _EB)],
                            s_buf.at[pl.ds(0, _EB)])
            pltpu.sync_copy(r_hbm.at[pl.ds(ebase, _EB)],
                            r_buf.at[pl.ds(0, _EB)])
            pltpu.sync_copy(ef_hbm.at[pl.ds(bstart, 1)],
                            ef_buf.at[pl.ds(0, 1)])
            make_step(in_sets[0], _EB // _LANES)
            pltpu.sync_copy(iout_buf.at[pl.ds(0, 2 * _EB)],
                            i_hbm.at[pl.ds(bstart * 2 * _EB, 2 * _EB)])
            pltpu.sync_copy(vout_buf.at[pl.ds(0, 2 * _EB)],
                            v_hbm.at[pl.ds(bstart * 2 * _EB, 2 * _EB)])

        pltpu.sync_copy(acc_buf, part_hbm.at[pl.ds(wid * n2, n2)])

    return sc_kernel


def _tc_extract_kernel(n_nodes, n_node_pad, d, row_block=2048):
    grid = (n_node_pad // row_block,)

    def body(vn_ref, re_ref, im_ref):
        rows = lax.broadcasted_iota(jnp.int32, (2, d), 0)
        cols = lax.broadcasted_iota(jnp.int32, (2, d), 1)
        sel = (rows == cols).astype(jnp.float32)
        prod = lax.dot_general(sel, vn_ref[...], (((1,), (1,)), ((), ())),
                               precision=lax.Precision.HIGHEST,
                               preferred_element_type=jnp.float32)
        re_ref[...] = prod[0, :]
        im_ref[...] = prod[1, :]

    return pl.pallas_call(
        body,
        grid=grid,
        in_specs=[pl.BlockSpec((row_block, d), lambda k: (k, 0))],
        out_specs=(pl.BlockSpec((row_block,), lambda k: (k,)),
                   pl.BlockSpec((row_block,), lambda k: (k,))),
        out_shape=(
            jax.ShapeDtypeStruct((n_node_pad,), jnp.float32),
            jax.ShapeDtypeStruct((n_node_pad,), jnp.float32),
        ),
    )


def _tc_matmul_kernel(n_nodes, n_pad, d, out_dim, row_block):
    grid = (n_pad // row_block,)
    n2 = 2 * n_pad

    def body(vn_ref, parts_ref, w1_ref, w2_ref, b_ref, out_ref):
        k = pl.program_id(0)
        net_re = jnp.zeros((row_block,), jnp.float32)
        net_im = jnp.zeros((row_block,), jnp.float32)
        for w in range(_NW):
            net_re = net_re + parts_ref[pl.ds(w * n2 + k * row_block,
                                              row_block)]
            net_im = net_im + parts_ref[pl.ds(w * n2 + n_pad + k * row_block,
                                              row_block)]
        net = jnp.stack([net_re, net_im], axis=0)
        acc = jnp.dot(vn_ref[...], w1_ref[...],
                      preferred_element_type=jnp.float32)
        acc = acc + lax.dot_general(
            net, w2_ref[...], (((0,), (0,)), ((), ())),
            preferred_element_type=jnp.float32)
        acc = acc + b_ref[...]
        out_ref[...] = jnp.maximum(acc, 0.0)

    return pl.pallas_call(
        body,
        grid=grid,
        in_specs=[
            pl.BlockSpec((row_block, d), lambda k: (k, 0)),
            pl.BlockSpec((_NW * n2,), lambda k: (0,)),
            pl.BlockSpec((d, out_dim), lambda k: (0, 0)),
            pl.BlockSpec((2, out_dim), lambda k: (0, 0)),
            pl.BlockSpec((1, out_dim), lambda k: (0, 0)),
        ],
        out_specs=pl.BlockSpec((row_block, out_dim), lambda k: (k, 0)),
        out_shape=jax.ShapeDtypeStruct((n_nodes, out_dim), jnp.float32),
    )


def kernel(V_node, senders, receivers, edge_features, W, b):
    n_nodes, d = V_node.shape
    n_edges = senders.shape[0]
    de = edge_features.shape[1]
    out_dim = W.shape[1]
    nb = n_edges // _EB

    row_block = 1024
    n_pad = -(-n_nodes // row_block) * row_block

    efb = edge_features.reshape(nb, _EB, de).transpose(0, 2, 1)

    v2_re, v2_im = _tc_extract_kernel(n_nodes, n_pad, d)(V_node)

    sc = _sc_edge_kernel(n_nodes, n_pad, n_pad, n_edges, cb=39)
    zeros = jnp.zeros((2 * n_pad,), jnp.float32)
    i_blk, v_blk, parts = sc(v2_re, v2_im, zeros, efb, senders, receivers)

    I_edge = (i_blk.reshape(nb, 2, _EB).transpose(0, 2, 1)
              .reshape(n_edges, 2))
    V_edge = (v_blk.reshape(nb, 2, _EB).transpose(0, 2, 1)
              .reshape(n_edges, 2))

    tc = _tc_matmul_kernel(n_nodes, n_pad, d, out_dim, 2048)
    V_out = tc(V_node, parts, W[:d], W[d:], b.reshape(1, out_dim))
    return (V_out, I_edge, V_edge)


# speedup vs baseline: 1.0497x; 1.0497x over previous
import functools

import jax
import jax.numpy as jnp
from jax import lax
from jax.experimental import pallas as pl
from jax.experimental.pallas import tpu as pltpu
from jax.experimental.pallas import tpu_sc as plsc

_NC = 2
_NS = 16
_NW = _NC * _NS
_LANES = 16
_EB = 128


def _sc_edge_kernel(n_nodes, n_node_pad, n_pad, n_edges, cb):
    nb = n_edges // _EB
    base_nb = nb // _NW
    extra = nb - base_nb * _NW
    n_chunks = base_nb // cb
    assert base_nb == n_chunks * cb
    chunk = cb * _EB
    n2 = 2 * n_pad

    mesh = plsc.VectorSubcoreMesh(core_axis_name="c", subcore_axis_name="s")

    edge_bufs = (
        pltpu.VMEM((chunk,), jnp.int32),
        pltpu.VMEM((chunk,), jnp.int32),
        pltpu.VMEM((cb, 4, _EB), jnp.float32),
        pltpu.VMEM((2 * chunk,), jnp.float32),
        pltpu.VMEM((2 * chunk,), jnp.float32),
        pltpu.SemaphoreType.DMA,
        pltpu.SemaphoreType.DMA,
    )

    @functools.partial(
        pl.kernel,
        mesh=mesh,
        compiler_params=pltpu.CompilerParams(needs_layout_passes=False),
        out_type=(
            jax.ShapeDtypeStruct((2 * n_edges,), jnp.float32),
            jax.ShapeDtypeStruct((2 * n_edges,), jnp.float32),
            jax.ShapeDtypeStruct((_NW * n2,), jnp.float32),
        ),
        name="sc_edges",
        scratch_types=(
            pltpu.VMEM((n_node_pad,), jnp.float32),
            pltpu.VMEM((n_node_pad,), jnp.float32),
            pltpu.VMEM((n2,), jnp.float32),
            pltpu.SemaphoreType.DMA,
        ) + edge_bufs + edge_bufs,
    )
    def sc_kernel(vre_hbm, vim_hbm, zeros_hbm, ef_hbm, s_hbm, r_hbm,
                  i_hbm, v_hbm, part_hbm,
                  vre_buf, vim_buf, acc_buf, v2_sem, *bufs):
        sets = (bufs[:7], bufs[7:])
        wid = lax.axis_index("s") * _NC + lax.axis_index("c")
        iota = lax.iota(jnp.int32, _LANES)
        start_block = wid * base_nb + jnp.minimum(wid, extra)
        total = n_chunks

        v2_d = [pltpu.async_copy(vre_hbm, vre_buf, v2_sem),
                pltpu.async_copy(vim_hbm, vim_buf, v2_sem),
                pltpu.async_copy(zeros_hbm, acc_buf, v2_sem)]

        def issue_in(c, bset):
            s_buf, r_buf, ef_buf = bset[0], bset[1], bset[2]
            bstart = start_block + c * cb
            ebase = bstart * _EB
            sem = bset[5]
            return [
                pltpu.async_copy(s_hbm.at[pl.ds(ebase, chunk)], s_buf, sem),
                pltpu.async_copy(r_hbm.at[pl.ds(ebase, chunk)], r_buf, sem),
                pltpu.async_copy(ef_hbm.at[pl.ds(bstart, cb)], ef_buf, sem),
            ]

        def issue_out(c, bset):
            iout_buf, vout_buf = bset[3], bset[4]
            bstart = start_block + c * cb
            sem = bset[6]
            return [
                pltpu.async_copy(iout_buf,
                                 i_hbm.at[pl.ds(bstart * 2 * _EB, 2 * chunk)],
                                 sem),
                pltpu.async_copy(vout_buf,
                                 v_hbm.at[pl.ds(bstart * 2 * _EB, 2 * chunk)],
                                 sem),
            ]

        def make_step(bset, nsteps):
            s_buf, r_buf, ef_buf, iout_buf, vout_buf = bset[:5]

            @plsc.parallel_loop(0, nsteps * _LANES, step=_LANES, unroll=4)
            def step(off):
                blk = off // _EB
                rem = off - blk * _EB
                s = s_buf[pl.ds(off, _LANES)]
                r = r_buf[pl.ds(off, _LANES)]
                g = ef_buf[blk, 0, pl.ds(rem, _LANES)]
                bb = ef_buf[blk, 1, pl.ds(rem, _LANES)]
                vr_re = plsc.load_gather(vre_buf, [r])
                vr_im = plsc.load_gather(vim_buf, [r])
                vs_re = plsc.load_gather(vre_buf, [s])
                vs_im = plsc.load_gather(vim_buf, [s])
                v_re = vr_re - vs_re
                v_im = vr_im - vs_im
                i_re = g * v_re - bb * v_im
                i_im = g * v_im + bb * v_re
                pos = blk * 2 * _EB + rem
                vout_buf[pl.ds(pos, _LANES)] = v_re
                vout_buf[pl.ds(pos + _EB, _LANES)] = v_im
                iout_buf[pl.ds(pos, _LANES)] = i_re
                iout_buf[pl.ds(pos + _EB, _LANES)] = i_im
                plsc.addupdate_scatter(acc_buf, [r], i_re)
                plsc.addupdate_scatter(acc_buf, [r + n_pad], i_im)
                plsc.addupdate_scatter(acc_buf, [s], -i_re)
                plsc.addupdate_scatter(acc_buf, [s + n_pad], -i_im)

        in_d = {0: issue_in(0, sets[0])}
        for d in v2_d:
            d.wait()

        out_d = {}
        for c in range(total):
            bset = sets[c % 2]
            for d in in_d.pop(c):
                d.wait()
            if c + 1 < total:
                in_d[c + 1] = issue_in(c + 1, sets[(c + 1) % 2])
            if c - 2 in out_d:
                for d in out_d.pop(c - 2):
                    d.wait()
            make_step(bset, chunk // _LANES)
            out_d[c] = issue_out(c, bset)

        for c in sorted(out_d):
            for d in out_d.pop(c):
                d.wait()

        @pl.when(wid < extra)
        def _():
            bset = sets[0]
            s_buf, r_buf, ef_buf, iout_buf, vout_buf = bset[:5]
            bstart = start_block + base_nb
            ebase = bstart * _EB
            pltpu.sync_copy(s_hbm.at[pl.ds(ebase, _EB)],
                            s_buf.at[pl.ds(0, _EB)])
            pltpu.sync_copy(r_hbm.at[pl.ds(ebase, _EB)],
                            r_buf.at[pl.ds(0, _EB)])
            pltpu.sync_copy(ef_hbm.at[pl.ds(bstart, 1)],
                            ef_buf.at[pl.ds(0, 1)])
            make_step(bset, _EB // _LANES)
            pltpu.sync_copy(iout_buf.at[pl.ds(0, 2 * _EB)],
                            i_hbm.at[pl.ds(bstart * 2 * _EB, 2 * _EB)])
            pltpu.sync_copy(vout_buf.at[pl.ds(0, 2 * _EB)],
                            v_hbm.at[pl.ds(bstart * 2 * _EB, 2 * _EB)])

        pltpu.sync_copy(acc_buf, part_hbm.at[pl.ds(wid * n2, n2)])

    return sc_kernel


def _tc_extract_kernel(n_nodes, n_node_pad, d, row_block=2048):
    grid = (n_node_pad // row_block,)

    def body(vn_ref, re_ref, im_ref):
        rows = lax.broadcasted_iota(jnp.int32, (2, d), 0)
        cols = lax.broadcasted_iota(jnp.int32, (2, d), 1)
        sel = (rows == cols).astype(jnp.float32)
        prod = lax.dot_general(sel, vn_ref[...], (((1,), (1,)), ((), ())),
                               precision=lax.Precision.HIGHEST,
                               preferred_element_type=jnp.float32)
        re_ref[...] = prod[0, :]
        im_ref[...] = prod[1, :]

    return pl.pallas_call(
        body,
        grid=grid,
        in_specs=[pl.BlockSpec((row_block, d), lambda k: (k, 0))],
        out_specs=(pl.BlockSpec((row_block,), lambda k: (k,)),
                   pl.BlockSpec((row_block,), lambda k: (k,))),
        out_shape=(
            jax.ShapeDtypeStruct((n_node_pad,), jnp.float32),
            jax.ShapeDtypeStruct((n_node_pad,), jnp.float32),
        ),
    )


def _tc_matmul_kernel(n_nodes, n_pad, d, out_dim, row_block):
    grid = (n_pad // row_block,)
    n2 = 2 * n_pad

    def body(vn_ref, parts_ref, w1_ref, w2_ref, b_ref, out_ref):
        k = pl.program_id(0)
        net_re = jnp.zeros((row_block,), jnp.float32)
        net_im = jnp.zeros((row_block,), jnp.float32)
        for w in range(_NW):
            net_re = net_re + parts_ref[pl.ds(w * n2 + k * row_block,
                                              row_block)]
            net_im = net_im + parts_ref[pl.ds(w * n2 + n_pad + k * row_block,
                                              row_block)]
        net = jnp.stack([net_re, net_im], axis=0)
        acc = jnp.dot(vn_ref[...], w1_ref[...],
                      preferred_element_type=jnp.float32)
        acc = acc + lax.dot_general(
            net, w2_ref[...], (((0,), (0,)), ((), ())),
            preferred_element_type=jnp.float32)
        acc = acc + b_ref[...]
        out_ref[...] = jnp.maximum(acc, 0.0)

    return pl.pallas_call(
        body,
        grid=grid,
        in_specs=[
            pl.BlockSpec((row_block, d), lambda k: (k, 0)),
            pl.BlockSpec((_NW * n2,), lambda k: (0,)),
            pl.BlockSpec((d, out_dim), lambda k: (0, 0)),
            pl.BlockSpec((2, out_dim), lambda k: (0, 0)),
            pl.BlockSpec((1, out_dim), lambda k: (0, 0)),
        ],
        out_specs=pl.BlockSpec((row_block, out_dim), lambda k: (k, 0)),
        out_shape=jax.ShapeDtypeStruct((n_nodes, out_dim), jnp.float32),
    )


def kernel(V_node, senders, receivers, edge_features, W, b):
    n_nodes, d = V_node.shape
    n_edges = senders.shape[0]
    de = edge_features.shape[1]
    out_dim = W.shape[1]
    nb = n_edges // _EB

    row_block = 1024
    n_pad = -(-n_nodes // row_block) * row_block

    efb = edge_features.reshape(nb, _EB, de).transpose(0, 2, 1)

    v2_re, v2_im = _tc_extract_kernel(n_nodes, n_pad, d)(V_node)

    sc = _sc_edge_kernel(n_nodes, n_pad, n_pad, n_edges, cb=26)
    zeros = jnp.zeros((2 * n_pad,), jnp.float32)
    i_blk, v_blk, parts = sc(v2_re, v2_im, zeros, efb, senders, receivers)

    I_edge = (i_blk.reshape(nb, 2, _EB).transpose(0, 2, 1)
              .reshape(n_edges, 2))
    V_edge = (v_blk.reshape(nb, 2, _EB).transpose(0, 2, 1)
              .reshape(n_edges, 2))

    tc = _tc_matmul_kernel(n_nodes, n_pad, d, out_dim, 2048)
    V_out = tc(V_node, parts, W[:d], W[d:], b.reshape(1, out_dim))
    return (V_out, I_edge, V_edge)
